# baseline (speedup 1.0000x reference)
import jax
import jax.numpy as jnp
from jax.experimental import pallas as pl
from jax.experimental.pallas import tpu as pltpu

_T = 9.210440366976517


def _focal_terms(x, g, v):
    xc = jnp.clip(x, -_T, _T)
    e = jnp.exp(xc)
    one_m_p = 1.0 / (1.0 + e)
    p = e * one_m_p
    sp = jnp.log1p(e)
    log_p = xc - sp
    log_1mp = -sp
    posf = (g == 1.0).astype(jnp.float32)
    negf = (g < 1.0).astype(jnp.float32)
    w = 1.0 - g
    w2 = w * w
    neg_w = w2 * w2
    s = jnp.sum((log_p * one_m_p * one_m_p * posf
                 + log_1mp * p * p * neg_w * negf) * v)
    return s, jnp.sum(posf)


def _make_body(nsteps, B, K, HW):
    def body(ht, hb, gt, gb, valt, valb,
             a0, ga0, a1, ga1, a2, ga2,
             vt, vb, indt, indb, mk, got, gob,
             out, acc):
        i = pl.program_id(0)

        @pl.when(i == 0)
        def _init():
            acc[0] = 0.0
            acc[1] = 0.0
            acc[2] = 0.0
            acc[3] = 0.0

        s_tl, n_tl = _focal_terms(ht[...], gt[...], valt[...])
        s_br, n_br = _focal_terms(hb[...], gb[...], valb[...])
        acc[0] = acc[0] + s_tl
        acc[1] = acc[1] + n_tl
        acc[2] = acc[2] + s_br
        acc[3] = acc[3] + n_br

        @pl.when(i == nsteps - 1)
        def _final():
            def focal(x, g):
                s, n = _focal_terms(x, g, 1.0)
                return -s / n

            att = (focal(a0[...], ga0[...])
                   + focal(a1[...], ga1[...])
                   + focal(a2[...], ga2[...]))

            mkf = mk[...]
            num_tot = jnp.sum(mkf)
            iota = jax.lax.broadcasted_iota(jnp.int32, (K, HW), 1)
            pull = 0.0
            osum = 0.0
            for b in range(B):
                oh_t = (indt[b, :][:, None] == iota).astype(jnp.float32)
                oh_b = (indb[b, :][:, None] == iota).astype(jnp.float32)
                gv_t = jax.lax.dot_general(
                    oh_t, vt[b], (((1,), (1,)), ((), ())),
                    preferred_element_type=jnp.float32)
                gv_b = jax.lax.dot_general(
                    oh_b, vb[b], (((1,), (1,)), ((), ())),
                    preferred_element_type=jnp.float32)
                mb = mkf[b, :]
                nb = jnp.sum(mb)
                dtag = gv_t[:, 0] - gv_b[:, 0]
                pull = pull + jnp.sum(dtag * dtag * 0.5 / (nb + 1e-4) * mb)
                for gv, go in ((gv_t, got), (gv_b, gob)):
                    for c in range(2):
                        d = gv[:, 1 + c] - go[b, c, :]
                        ad = jnp.abs(d)
                        l = jnp.where(ad < 1.0, 0.5 * d * d, ad - 0.5)
                        osum = osum + jnp.sum(l * mb)

            big = -acc[0] / acc[1] - acc[2] / acc[3]
            total = big + att + pull + osum / (num_tot + 1e-4)
            out[...] = jnp.broadcast_to(total, (1, 1))

    return body


def kernel(tl_heat, br_heat, tl_tag, br_tag, tl_off, br_off,
           att0, att1, att2, gt_tl_heat, gt_br_heat, gt_mask,
           gt_tl_off, gt_br_off, gt_tl_ind, gt_br_ind,
           gt_tl_valid, gt_br_valid, gt_att0, gt_att1, gt_att2):
    B, C, H, W = tl_heat.shape
    K = gt_mask.shape[1]
    HW = H * W
    R = B * C
    ROWS = 64
    nsteps = R // ROWS

    big = [a.reshape(R, HW) for a in
           (tl_heat, br_heat, gt_tl_heat, gt_br_heat,
            gt_tl_valid, gt_br_valid)]
    a0 = att0.reshape(B, -1)
    ga0 = gt_att0.reshape(B, -1)
    a1 = att1.reshape(B, -1)
    ga1 = gt_att1.reshape(B, -1)
    a2 = att2.reshape(B, -1)
    ga2 = gt_att2.reshape(B, -1)
    vals_tl = jnp.concatenate(
        [tl_tag.reshape(B, 1, HW), tl_off.reshape(B, 2, HW)], axis=1)
    vals_br = jnp.concatenate(
        [br_tag.reshape(B, 1, HW), br_off.reshape(B, 2, HW)], axis=1)
    ind_tl = gt_tl_ind.astype(jnp.int32)
    ind_br = gt_br_ind.astype(jnp.int32)
    maskf = gt_mask.astype(jnp.float32)
    goff_tl = jnp.transpose(gt_tl_off, (0, 2, 1))
    goff_br = jnp.transpose(gt_br_off, (0, 2, 1))

    big_spec = pl.BlockSpec((ROWS, HW), lambda i: (i, 0))
    full2 = lambda shape: pl.BlockSpec(shape, lambda i: (0, 0))
    full3 = lambda shape: pl.BlockSpec(shape, lambda i: (0, 0, 0))

    res = pl.pallas_call(
        _make_body(nsteps, B, K, HW),
        grid=(nsteps,),
        in_specs=[big_spec] * 6 + [
            full2(a0.shape), full2(ga0.shape),
            full2(a1.shape), full2(ga1.shape),
            full2(a2.shape), full2(ga2.shape),
            full3(vals_tl.shape), full3(vals_br.shape),
            full2(ind_tl.shape), full2(ind_br.shape),
            full2(maskf.shape),
            full3(goff_tl.shape), full3(goff_br.shape),
        ],
        out_specs=pl.BlockSpec((1, 1), lambda i: (0, 0)),
        out_shape=jax.ShapeDtypeStruct((1, 1), jnp.float32),
        scratch_shapes=[pltpu.SMEM((4,), jnp.float32)],
        compiler_params=pltpu.CompilerParams(
            dimension_semantics=("arbitrary",)),
    )(*big, a0, ga0, a1, ga1, a2, ga2,
      vals_tl, vals_br, ind_tl, ind_br, maskf, goff_tl, goff_br)
    return res.reshape(1)
